# baseline (speedup 1.0000x reference)
import functools

import jax
import jax.numpy as jnp
import numpy as np
from jax import lax
from jax.experimental import pallas as pl
from jax.experimental.pallas import tpu as pltpu
from jax.experimental.pallas import tpu_sc as plsc

NUM_EMB = 1000000
DIM = 64
QMAX = 127.0
BATCH = 16384
HIST = 50


_ROWS_PER_BLK = 8000


def _scale_body(w_ref, out_ref):
    i = pl.program_id(0)
    m = jnp.max(jnp.abs(w_ref[...]))

    @pl.when(i == 0)
    def _init():
        out_ref[0, 0] = m

    @pl.when(i > 0)
    def _acc():
        out_ref[0, 0] = jnp.maximum(out_ref[0, 0], m)

    @pl.when(i == pl.num_programs(0) - 1)
    def _fin():
        out_ref[0, 0] = jnp.maximum(out_ref[0, 0] / QMAX, 1e-8)


def _compute_scale(weight):
    return pl.pallas_call(
        _scale_body,
        grid=(NUM_EMB // _ROWS_PER_BLK,),
        in_specs=[pl.BlockSpec((_ROWS_PER_BLK, DIM), lambda i: (i, 0))],
        out_specs=pl.BlockSpec(memory_space=pltpu.SMEM),
        out_shape=jax.ShapeDtypeStruct((1, 1), jnp.float32),
    )(weight)


_B = BATCH * HIST
_NW = 32
_B_PER_W = _B // _NW
_CB = 8
_CHUNK = _CB * HIST
_NCHUNK = _B_PER_W // _CHUNK

_SIGN_MASK = np.uint32(0x80000000)
_MAGIC_BITS = np.uint32(0x4B000000)


def _gather_fq(table, idx_flat, scale_vec):
    mesh = plsc.VectorSubcoreMesh(core_axis_name="c", subcore_axis_name="s")

    @functools.partial(
        pl.kernel,
        mesh=mesh,
        out_type=jax.ShapeDtypeStruct((BATCH, HIST, DIM), jnp.float32),
        scratch_types=[
            pltpu.VMEM((2, _CHUNK), jnp.int32),
            [pltpu.VMEM((_CHUNK, DIM), jnp.float32) for _ in range(2)],
            [pltpu.VMEM((_CB, HIST, DIM), jnp.float32) for _ in range(2)],
            pltpu.VMEM((16,), jnp.float32),
            [pltpu.SemaphoreType.DMA for _ in range(2)],
            [pltpu.SemaphoreType.DMA for _ in range(2)],
        ],
        compiler_params=pltpu.CompilerParams(use_tc_tiling_on_sc=False,
                                             needs_layout_passes=False),
    )
    def k(table_hbm, idx_hbm, scale_hbm, out_hbm, idx_v, rin, rout, scale_v,
          sem_g, sem_o):
        wid = lax.axis_index("s") * 2 + lax.axis_index("c")
        base = wid * _B_PER_W
        bbase = wid * (_B_PER_W // HIST)
        pltpu.sync_copy(scale_hbm, scale_v)
        s = scale_v[...]
        rs = 1.0 / s

        def dequant(b):
            def batch_body(r1, _):
                def hist_body(r2, _):
                    r = r1 * HIST + r2
                    for c in range(DIM // 16):
                        v = rin[b][r, pl.ds(c * 16, 16)]
                        t = v * rs
                        tb = plsc.bitcast(t, jnp.uint32)
                        csign = plsc.bitcast(
                            (tb & _SIGN_MASK) | _MAGIC_BITS, jnp.float32)
                        q = (t + csign) - csign
                        rout[b][r1, r2, pl.ds(c * 16, 16)] = q * s
                    return 0

                lax.fori_loop(0, HIST, hist_body, 0, unroll=False)
                return 0

            lax.fori_loop(0, _CB, batch_body, 0, unroll=False)

        def start_gather(b, j):
            off = base + j * _CHUNK
            pltpu.sync_copy(idx_hbm.at[pl.ds(off, _CHUNK)], idx_v.at[b])
            pltpu.async_copy(table_hbm.at[idx_v.at[b]], rin[b], sem_g[b])

        for b in range(2):
            start_gather(b, b)

        def pair_body(p, _):
            for b in range(2):
                j = 2 * p + b
                boff = bbase + j * _CB
                pltpu.make_async_copy(table_hbm.at[idx_v.at[b]], rin[b],
                                      sem_g[b]).wait()
                dequant(b)
                pltpu.async_copy(rout[b], out_hbm.at[pl.ds(boff, _CB)],
                                 sem_o[b])

                @pl.when(p < _NCHUNK // 2 - 1)
                def _prefetch():
                    pltpu.make_async_copy(rout[b],
                                          out_hbm.at[pl.ds(bbase, _CB)],
                                          sem_o[b]).wait()
                    start_gather(b, j + 2)

            return 0

        lax.fori_loop(0, _NCHUNK // 2, pair_body, 0, unroll=False)

        for b in range(2):
            pltpu.make_async_copy(rout[b], out_hbm.at[pl.ds(bbase, _CB)],
                                  sem_o[b]).wait()

    return k(table, idx_flat, scale_vec)


def kernel(x, weight):
    scale = _compute_scale(weight)
    scale_vec = jnp.broadcast_to(scale.reshape(()), (16,))
    return _gather_fq(weight, x.reshape(-1), scale_vec)


# speedup vs baseline: 1.0218x; 1.0218x over previous
import functools

import jax
import jax.numpy as jnp
import numpy as np
from jax import lax
from jax.experimental import pallas as pl
from jax.experimental.pallas import tpu as pltpu
from jax.experimental.pallas import tpu_sc as plsc

NUM_EMB = 1000000
DIM = 64
QMAX = 127.0
BATCH = 16384
HIST = 50


_ROWS_PER_BLK = 8000


def _scale_body(w_ref, out_ref):
    i = pl.program_id(0)
    m = jnp.max(jnp.abs(w_ref[...]))

    @pl.when(i == 0)
    def _init():
        out_ref[0, 0] = m

    @pl.when(i > 0)
    def _acc():
        out_ref[0, 0] = jnp.maximum(out_ref[0, 0], m)

    @pl.when(i == pl.num_programs(0) - 1)
    def _fin():
        out_ref[0, 0] = jnp.maximum(out_ref[0, 0] / QMAX, 1e-8)


def _compute_scale(weight):
    return pl.pallas_call(
        _scale_body,
        grid=(NUM_EMB // _ROWS_PER_BLK,),
        in_specs=[pl.BlockSpec((_ROWS_PER_BLK, DIM), lambda i: (i, 0))],
        out_specs=pl.BlockSpec(memory_space=pltpu.SMEM),
        out_shape=jax.ShapeDtypeStruct((1, 1), jnp.float32),
    )(weight)


_B = BATCH * HIST
_NW = 32
_B_PER_W = _B // _NW
_CHUNK = 400
_NCHUNK = _B_PER_W // _CHUNK

_SIGN_MASK = np.uint32(0x80000000)
_MAGIC_BITS = np.uint32(0x4B000000)


def _gather_fq(table, idx_flat, scale_vec):
    mesh = plsc.VectorSubcoreMesh(core_axis_name="c", subcore_axis_name="s")

    @functools.partial(
        pl.kernel,
        mesh=mesh,
        out_type=jax.ShapeDtypeStruct((_B // 2, 2 * DIM), jnp.float32),
        scratch_types=[
            pltpu.VMEM((2, _CHUNK), jnp.int32),
            [pltpu.VMEM((_CHUNK, DIM), jnp.float32) for _ in range(2)],
            [pltpu.VMEM((_CHUNK // 2, 2 * DIM), jnp.float32)
             for _ in range(2)],
            pltpu.VMEM((16,), jnp.float32),
            [pltpu.SemaphoreType.DMA for _ in range(2)],
            [pltpu.SemaphoreType.DMA for _ in range(2)],
        ],
        compiler_params=pltpu.CompilerParams(use_tc_tiling_on_sc=False,
                                             needs_layout_passes=False),
    )
    def k(table_hbm, idx_hbm, scale_hbm, out_hbm, idx_v, rin, rout, scale_v,
          sem_g, sem_o):
        wid = lax.axis_index("s") * 2 + lax.axis_index("c")
        base = wid * _B_PER_W
        base2 = wid * (_B_PER_W // 2)
        pltpu.sync_copy(scale_hbm, scale_v)
        s = scale_v[...]
        rs = 1.0 / s

        def dequant(b):
            def row_body(r, _):
                for c in range(DIM // 16):
                    v = rin[b][r, pl.ds(c * 16, 16)]
                    t = v * rs
                    tb = plsc.bitcast(t, jnp.uint32)
                    csign = plsc.bitcast((tb & _SIGN_MASK) | _MAGIC_BITS,
                                         jnp.float32)
                    q = (t + csign) - csign
                    rout[b][r >> 1, pl.ds((r & 1) * DIM + c * 16, 16)] = q * s
                return 0

            lax.fori_loop(0, _CHUNK, row_body, 0, unroll=False)

        def start_gather(b, j):
            off = base + j * _CHUNK
            pltpu.sync_copy(idx_hbm.at[pl.ds(off, _CHUNK)], idx_v.at[b])
            pltpu.async_copy(table_hbm.at[idx_v.at[b]], rin[b], sem_g[b])

        for b in range(2):
            start_gather(b, b)

        def pair_body(p, _):
            for b in range(2):
                j = 2 * p + b
                off2 = base2 + j * (_CHUNK // 2)
                pltpu.make_async_copy(table_hbm.at[idx_v.at[b]], rin[b],
                                      sem_g[b]).wait()
                dequant(b)
                pltpu.async_copy(rout[b],
                                 out_hbm.at[pl.ds(off2, _CHUNK // 2)],
                                 sem_o[b])

                @pl.when(p < _NCHUNK // 2 - 1)
                def _prefetch():
                    pltpu.make_async_copy(
                        rout[b], out_hbm.at[pl.ds(base2, _CHUNK // 2)],
                        sem_o[b]).wait()
                    start_gather(b, j + 2)

            return 0

        lax.fori_loop(0, _NCHUNK // 2, pair_body, 0, unroll=False)

        for b in range(2):
            pltpu.make_async_copy(rout[b],
                                  out_hbm.at[pl.ds(base2, _CHUNK // 2)],
                                  sem_o[b]).wait()

    return k(table, idx_flat, scale_vec)


def kernel(x, weight):
    scale = _compute_scale(weight)
    scale_vec = jnp.broadcast_to(scale.reshape(()), (16,))
    out = _gather_fq(weight, x.reshape(-1), scale_vec)
    return out.reshape(BATCH, HIST, DIM)


# speedup vs baseline: 1.0565x; 1.0340x over previous
import functools

import jax
import jax.numpy as jnp
import numpy as np
from jax import lax
from jax.experimental import pallas as pl
from jax.experimental.pallas import tpu as pltpu
from jax.experimental.pallas import tpu_sc as plsc

NUM_EMB = 1000000
DIM = 64
QMAX = 127.0
BATCH = 16384
HIST = 50

_B = BATCH * HIST
_NW = 32
_B_PER_W = _B // _NW
_CHUNK = 400
_NCHUNK = _B_PER_W // _CHUNK

_SROWS = NUM_EMB // 16
_SCH = 156
_STAIL = _SROWS - _SCH * _CHUNK

_SIGN_MASK = np.uint32(0x80000000)
_MAGIC_BITS = np.uint32(0x4B000000)


def _gather_fq(table, idx_flat):
    mesh = plsc.VectorSubcoreMesh(core_axis_name="c", subcore_axis_name="s")

    @functools.partial(
        pl.kernel,
        mesh=mesh,
        out_type=jax.ShapeDtypeStruct((_B // 2, 2 * DIM), jnp.float32),
        scratch_types=[
            pltpu.VMEM((2, _CHUNK), jnp.int32),
            [pltpu.VMEM((_CHUNK, DIM), jnp.float32) for _ in range(2)],
            [pltpu.VMEM((_CHUNK // 2, 2 * DIM), jnp.float32)
             for _ in range(2)],
            pltpu.VMEM((16,), jnp.float32),
            pltpu.VMEM((16, 16), jnp.float32),
            pltpu.VMEM_SHARED((16, 16), jnp.float32),
            [pltpu.SemaphoreType.DMA for _ in range(2)],
            [pltpu.SemaphoreType.DMA for _ in range(2)],
        ],
        compiler_params=pltpu.CompilerParams(use_tc_tiling_on_sc=False,
                                             needs_layout_passes=False),
    )
    def k(table_hbm, idx_hbm, out_hbm, idx_v, rin, rout, red_v, redall_v,
          shared, sem_g, sem_o):
        cid = lax.axis_index("c")
        sid = lax.axis_index("s")
        wid = sid * 2 + cid
        base = wid * _B_PER_W
        base2 = wid * (_B_PER_W // 2)

        srow = sid * _SROWS

        def scan_rows(b, nrows, m):
            def row_body(r, m):
                for c in range(DIM // 16):
                    m = jnp.maximum(m, jnp.abs(rin[b][r, pl.ds(c * 16, 16)]))
                return m

            return lax.fori_loop(0, nrows, row_body, m, unroll=False)

        for b in range(2):
            pltpu.async_copy(table_hbm.at[pl.ds(srow + b * _CHUNK, _CHUNK)],
                             rin[b], sem_g[b])

        m = jnp.zeros((16,), jnp.float32)

        def scan_pair(p, m):
            for b in range(2):
                j = 2 * p + b
                pltpu.make_async_copy(
                    table_hbm.at[pl.ds(srow, _CHUNK)], rin[b],
                    sem_g[b]).wait()
                m = scan_rows(b, _CHUNK, m)

                @pl.when(p < _SCH // 2 - 1)
                def _prefetch():
                    pltpu.async_copy(
                        table_hbm.at[pl.ds(srow + (j + 2) * _CHUNK, _CHUNK)],
                        rin[b], sem_g[b])

            return m

        m = lax.fori_loop(0, _SCH // 2, scan_pair, m, unroll=False)

        pltpu.sync_copy(table_hbm.at[pl.ds(srow + _SCH * _CHUNK, _STAIL)],
                        rin[0].at[pl.ds(0, _STAIL)])
        m = scan_rows(0, _STAIL, m)

        red_v[...] = m
        pltpu.sync_copy(red_v, shared.at[sid])
        plsc.subcore_barrier()
        pltpu.sync_copy(shared, redall_v)
        for t in range(16):
            m = jnp.maximum(m, redall_v[t, :])
        absmax_v = jnp.full((16,), jnp.max(m), jnp.float32)
        s = jnp.maximum(absmax_v / QMAX, 1e-8)
        rs = 1.0 / s

        def dequant(b):
            def row_body(r, _):
                for c in range(DIM // 16):
                    v = rin[b][r, pl.ds(c * 16, 16)]
                    t = v * rs
                    tb = plsc.bitcast(t, jnp.uint32)
                    csign = plsc.bitcast((tb & _SIGN_MASK) | _MAGIC_BITS,
                                         jnp.float32)
                    q = (t + csign) - csign
                    rout[b][r >> 1, pl.ds((r & 1) * DIM + c * 16, 16)] = q * s
                return 0

            lax.fori_loop(0, _CHUNK, row_body, 0, unroll=False)

        def start_gather(b, j):
            off = base + j * _CHUNK
            pltpu.sync_copy(idx_hbm.at[pl.ds(off, _CHUNK)], idx_v.at[b])
            pltpu.async_copy(table_hbm.at[idx_v.at[b]], rin[b], sem_g[b])

        for b in range(2):
            start_gather(b, b)

        def pair_body(p, _):
            for b in range(2):
                j = 2 * p + b
                off2 = base2 + j * (_CHUNK // 2)
                pltpu.make_async_copy(table_hbm.at[idx_v.at[b]], rin[b],
                                      sem_g[b]).wait()
                dequant(b)
                pltpu.async_copy(rout[b],
                                 out_hbm.at[pl.ds(off2, _CHUNK // 2)],
                                 sem_o[b])

                @pl.when(p < _NCHUNK // 2 - 1)
                def _prefetch():
                    pltpu.make_async_copy(
                        rout[b], out_hbm.at[pl.ds(base2, _CHUNK // 2)],
                        sem_o[b]).wait()
                    start_gather(b, j + 2)

            return 0

        lax.fori_loop(0, _NCHUNK // 2, pair_body, 0, unroll=False)

        for b in range(2):
            pltpu.make_async_copy(rout[b],
                                  out_hbm.at[pl.ds(base2, _CHUNK // 2)],
                                  sem_o[b]).wait()

    return k(table, idx_flat)


def kernel(x, weight):
    out = _gather_fq(weight, x.reshape(-1))
    return out.reshape(BATCH, HIST, DIM)


# speedup vs baseline: 1.6393x; 1.5516x over previous
import functools

import jax
import jax.numpy as jnp
import numpy as np
from jax import lax
from jax.experimental import pallas as pl
from jax.experimental.pallas import tpu as pltpu
from jax.experimental.pallas import tpu_sc as plsc

NUM_EMB = 1000000
DIM = 64
QMAX = 127.0
BATCH = 16384
HIST = 50

_B = BATCH * HIST
_NW = 32
_B_PER_W = _B // _NW
_CHUNK = 400
_NCHUNK = _B_PER_W // _CHUNK

_SROWS = NUM_EMB // 16
_SCH = 156
_STAIL = _SROWS - _SCH * _CHUNK

_SIGN_MASK = np.uint32(0x80000000)
_MAGIC_BITS = np.uint32(0x4B000000)


def _gather_fq(table, idx_flat):
    mesh = plsc.VectorSubcoreMesh(core_axis_name="c", subcore_axis_name="s")

    @functools.partial(
        pl.kernel,
        mesh=mesh,
        out_type=jax.ShapeDtypeStruct((_B // 2, 2 * DIM), jnp.float32),
        scratch_types=[
            pltpu.VMEM((2, _CHUNK), jnp.int32),
            [pltpu.VMEM((_CHUNK, DIM), jnp.float32) for _ in range(2)],
            [pltpu.VMEM((_CHUNK // 2, 2 * DIM), jnp.float32)
             for _ in range(2)],
            pltpu.VMEM((16,), jnp.float32),
            pltpu.VMEM((16, 16), jnp.float32),
            pltpu.VMEM_SHARED((16, 16), jnp.float32),
            [pltpu.SemaphoreType.DMA for _ in range(2)],
            [pltpu.SemaphoreType.DMA for _ in range(2)],
        ],
        compiler_params=pltpu.CompilerParams(use_tc_tiling_on_sc=False,
                                             needs_layout_passes=False),
    )
    def k(table_hbm, idx_hbm, out_hbm, idx_v, rin, rout, red_v, redall_v,
          shared, sem_g, sem_o):
        cid = lax.axis_index("c")
        sid = lax.axis_index("s")
        wid = sid * 2 + cid
        base = wid * _B_PER_W
        base2 = wid * (_B_PER_W // 2)

        srow = sid * _SROWS

        def scan_rows(b, nrows, accs):
            def row_body(r2, accs):
                new = []
                for j in range(2):
                    for c in range(DIM // 16):
                        a = accs[j * 4 + c]
                        v = rin[b][2 * r2 + j, pl.ds(c * 16, 16)]
                        new.append(jnp.maximum(a, jnp.abs(v)))
                return tuple(new)

            return lax.fori_loop(0, nrows // 2, row_body, accs,
                                 unroll=False)

        for b in range(2):
            pltpu.async_copy(table_hbm.at[pl.ds(srow + b * _CHUNK, _CHUNK)],
                             rin[b], sem_g[b])

        accs = tuple(jnp.zeros((16,), jnp.float32) for _ in range(8))

        def scan_pair(p, accs):
            for b in range(2):
                j = 2 * p + b
                pltpu.make_async_copy(
                    table_hbm.at[pl.ds(srow, _CHUNK)], rin[b],
                    sem_g[b]).wait()
                accs = scan_rows(b, _CHUNK, accs)

                @pl.when(p < _SCH // 2 - 1)
                def _prefetch():
                    pltpu.async_copy(
                        table_hbm.at[pl.ds(srow + (j + 2) * _CHUNK, _CHUNK)],
                        rin[b], sem_g[b])

            return accs

        accs = lax.fori_loop(0, _SCH // 2, scan_pair, accs, unroll=False)

        pltpu.sync_copy(table_hbm.at[pl.ds(srow + _SCH * _CHUNK, _STAIL)],
                        rin[0].at[pl.ds(0, _STAIL)])
        accs = scan_rows(0, _STAIL, accs)
        m = accs[0]
        for a in accs[1:]:
            m = jnp.maximum(m, a)

        red_v[...] = m
        pltpu.sync_copy(red_v, shared.at[sid])
        plsc.subcore_barrier()
        pltpu.sync_copy(shared, redall_v)
        for t in range(16):
            m = jnp.maximum(m, redall_v[t, :])
        absmax_v = jnp.full((16,), jnp.max(m), jnp.float32)
        s = jnp.maximum(absmax_v / QMAX, 1e-8)
        rs = 1.0 / s

        def dequant(b):
            def row_body(rr, _):
                for j in range(2):
                    for c in range(DIM // 16):
                        v = rin[b][2 * rr + j, pl.ds(c * 16, 16)]
                        t = v * rs
                        tb = plsc.bitcast(t, jnp.uint32)
                        csign = plsc.bitcast(
                            (tb & _SIGN_MASK) | _MAGIC_BITS, jnp.float32)
                        q = (t + csign) - csign
                        rout[b][rr, pl.ds(j * DIM + c * 16, 16)] = q * s
                return 0

            lax.fori_loop(0, _CHUNK // 2, row_body, 0, unroll=False)

        def start_gather(b, j):
            off = base + j * _CHUNK
            pltpu.sync_copy(idx_hbm.at[pl.ds(off, _CHUNK)], idx_v.at[b])
            pltpu.async_copy(table_hbm.at[idx_v.at[b]], rin[b], sem_g[b])

        for b in range(2):
            start_gather(b, b)

        def pair_body(p, _):
            for b in range(2):
                j = 2 * p + b
                off2 = base2 + j * (_CHUNK // 2)
                pltpu.make_async_copy(table_hbm.at[idx_v.at[b]], rin[b],
                                      sem_g[b]).wait()
                dequant(b)
                pltpu.async_copy(rout[b],
                                 out_hbm.at[pl.ds(off2, _CHUNK // 2)],
                                 sem_o[b])

                @pl.when(p < _NCHUNK // 2 - 1)
                def _prefetch():
                    pltpu.make_async_copy(
                        rout[b], out_hbm.at[pl.ds(base2, _CHUNK // 2)],
                        sem_o[b]).wait()
                    start_gather(b, j + 2)

            return 0

        lax.fori_loop(0, _NCHUNK // 2, pair_body, 0, unroll=False)

        for b in range(2):
            pltpu.make_async_copy(rout[b],
                                  out_hbm.at[pl.ds(base2, _CHUNK // 2)],
                                  sem_o[b]).wait()

    return k(table, idx_flat)


def kernel(x, weight):
    out = _gather_fq(weight, x.reshape(-1))
    return out.reshape(BATCH, HIST, DIM)
